# Initial kernel scaffold; baseline (speedup 1.0000x reference)
import functools

import jax
import jax.numpy as jnp
from jax import lax
from jax.experimental import pallas as pl
from jax.experimental.pallas import tpu as pltpu
from jax.experimental.pallas import tpu_sc as plsc

_B = 16384
_D = 64
_NRC = 26
_NKC = 26

_info = plsc.get_sparse_core_info()
_NC, _NS = _info.num_cores, _info.num_subcores
_NW = _NC * _NS

_U_PER_W = _B // _NW
_KB = _B * _NRC
_K_PER_W = _KB // _NW
_CHUNK = 1024
_NCHUNK = _K_PER_W // _CHUNK


def _sc_gather_body(uf_hbm, cf_hbm, ef_hbm, uidx_hbm, cidx_hbm, eidx_hbm,
                    u_out, c_out, e_out,
                    uidx_v, urows_v, idx_v, rows_v, sem):
    wid = lax.axis_index("s") * _NC + lax.axis_index("c")

    ub = wid * _U_PER_W
    pltpu.sync_copy(uidx_hbm.at[pl.ds(ub, _U_PER_W)], uidx_v)
    pltpu.async_copy(uf_hbm.at[uidx_v], urows_v, sem).wait()
    pltpu.sync_copy(urows_v, u_out.at[pl.ds(ub, _U_PER_W)])

    kb = wid * _K_PER_W

    def chunk(idx_hbm, table_hbm, out_hbm, c, carry):
        off = kb + c * _CHUNK
        pltpu.sync_copy(idx_hbm.at[pl.ds(off, _CHUNK)], idx_v)
        pltpu.async_copy(table_hbm.at[idx_v], rows_v, sem).wait()
        pltpu.sync_copy(rows_v, out_hbm.at[pl.ds(off, _CHUNK)])
        return carry

    lax.fori_loop(0, _NCHUNK,
                  functools.partial(chunk, cidx_hbm, cf_hbm, c_out), 0)
    lax.fori_loop(0, _NCHUNK,
                  functools.partial(chunk, eidx_hbm, ef_hbm, e_out), 0)


_sc_gather = functools.partial(
    pl.kernel,
    mesh=plsc.VectorSubcoreMesh(core_axis_name="c", subcore_axis_name="s"),
    out_type=[
        jax.ShapeDtypeStruct((_B, _D), jnp.float32),
        jax.ShapeDtypeStruct((_KB, _D), jnp.float32),
        jax.ShapeDtypeStruct((_KB, _D), jnp.float32),
    ],
    scratch_types=[
        pltpu.VMEM((_U_PER_W,), jnp.int32),
        pltpu.VMEM((_U_PER_W, _D), jnp.float32),
        pltpu.VMEM((_CHUNK,), jnp.int32),
        pltpu.VMEM((_CHUNK, _D), jnp.float32),
        pltpu.SemaphoreType.DMA,
    ],
)(_sc_gather_body)


def _leaky(x):
    return jnp.where(x >= 0, x, 0.1 * x)


def _softmax(x):
    m = jnp.max(x, axis=1, keepdims=True)
    e = jnp.exp(x - m)
    return e / jnp.sum(e, axis=1, keepdims=True)


_BLK = 1024


def _dense_body(u_ref, c_ref, e_ref, rc_ref, rk_ref, out_ref):
    u = u_ref[...]
    ctx = c_ref[...]
    ent = e_ref[...]
    rc = rc_ref[...]
    rk = rk_ref[...]

    scores = _softmax(_leaky(jnp.dot(u, rc, preferred_element_type=jnp.float32)))
    ctx_agg = jnp.sum(scores[:, :, None] * ctx, axis=1)
    uf = _leaky(ctx_agg + u)
    imp = _softmax(_leaky(jnp.dot(uf, rk, preferred_element_type=jnp.float32)))
    es = jnp.sum(ent * uf[:, None, :], axis=2)
    out_ref[...] = jnp.sum(imp * es, axis=1)


_dense = pl.pallas_call(
    _dense_body,
    grid=(_B // _BLK,),
    in_specs=[
        pl.BlockSpec((_BLK, _D), lambda i: (i, 0)),
        pl.BlockSpec((_BLK, _NRC, _D), lambda i: (i, 0, 0)),
        pl.BlockSpec((_BLK, _NKC, _D), lambda i: (i, 0, 0)),
        pl.BlockSpec((_D, _NRC), lambda i: (0, 0)),
        pl.BlockSpec((_D, _NKC), lambda i: (0, 0)),
    ],
    out_specs=pl.BlockSpec((_BLK,), lambda i: (i,)),
    out_shape=jax.ShapeDtypeStruct((_B,), jnp.float32),
)


def kernel(user_factors, context_factors, entity_factors, relation_c, relation_k,
           user, item, contexts_index, entities_index):
    del item
    u_rows, c_rows, e_rows = _sc_gather(
        user_factors, context_factors, entity_factors,
        user.astype(jnp.int32),
        contexts_index.reshape(-1).astype(jnp.int32),
        entities_index.reshape(-1).astype(jnp.int32),
    )
    return _dense(u_rows,
                  c_rows.reshape(_B, _NRC, _D),
                  e_rows.reshape(_B, _NKC, _D),
                  relation_c, relation_k)


# speedup vs baseline: 1.3762x; 1.3762x over previous
import functools

import jax
import jax.numpy as jnp
from jax import lax
from jax.experimental import pallas as pl
from jax.experimental.pallas import tpu as pltpu
from jax.experimental.pallas import tpu_sc as plsc

_B = 16384
_D = 64
_NRC = 26
_NKC = 26

_NC, _NS = 2, 16
_NW = _NC * _NS

_U_PER_W = _B // _NW
_KB = _B * _NRC
_K_PER_W = _KB // _NW
_CHUNK = 1024
_NCHUNK = _K_PER_W // _CHUNK


def _sc_gather_body(uf_hbm, cf_hbm, ef_hbm, uidx_hbm, cidx_hbm, eidx_hbm,
                    u_out, c_out, e_out,
                    uidx_v, urows_v, idx_v, rows_v, sem):
    wid = lax.axis_index("s") * _NC + lax.axis_index("c")

    ub = wid * _U_PER_W
    pltpu.sync_copy(uidx_hbm.at[pl.ds(ub, _U_PER_W)], uidx_v)
    pltpu.async_copy(uf_hbm.at[uidx_v], urows_v, sem).wait()
    pltpu.sync_copy(urows_v, u_out.at[pl.ds(ub, _U_PER_W)])

    kb = wid * _K_PER_W

    def chunk(idx_hbm, table_hbm, out_hbm, c, carry):
        off = kb + c * _CHUNK
        pltpu.sync_copy(idx_hbm.at[pl.ds(off, _CHUNK)], idx_v)
        pltpu.async_copy(table_hbm.at[idx_v], rows_v, sem).wait()
        pltpu.sync_copy(rows_v, out_hbm.at[pl.ds(off, _CHUNK)])
        return carry

    lax.fori_loop(0, _NCHUNK,
                  functools.partial(chunk, cidx_hbm, cf_hbm, c_out), 0)
    lax.fori_loop(0, _NCHUNK,
                  functools.partial(chunk, eidx_hbm, ef_hbm, e_out), 0)


@functools.cache
def _sc_gather():
    return pl.kernel(
        _sc_gather_body,
        mesh=plsc.VectorSubcoreMesh(core_axis_name="c", subcore_axis_name="s",
                                    num_cores=_NC, num_subcores=_NS),
        out_type=[
            jax.ShapeDtypeStruct((_B, _D), jnp.float32),
            jax.ShapeDtypeStruct((_KB, _D), jnp.float32),
            jax.ShapeDtypeStruct((_KB, _D), jnp.float32),
        ],
        scratch_types=[
            pltpu.VMEM((_U_PER_W,), jnp.int32),
            pltpu.VMEM((_U_PER_W, _D), jnp.float32),
            pltpu.VMEM((_CHUNK,), jnp.int32),
            pltpu.VMEM((_CHUNK, _D), jnp.float32),
            pltpu.SemaphoreType.DMA,
        ],
        compiler_params=pltpu.CompilerParams(use_tc_tiling_on_sc=False),
    )


def _leaky(x):
    return jnp.where(x >= 0, x, 0.1 * x)


def _softmax(x):
    m = jnp.max(x, axis=1, keepdims=True)
    e = jnp.exp(x - m)
    return e / jnp.sum(e, axis=1, keepdims=True)


_BLK = 512


def _dense_body(u_ref, c_ref, e_ref, rc_ref, rk_ref, out_ref):
    u = u_ref[...]
    ctx = c_ref[...]
    ent = e_ref[...]
    rc = rc_ref[...]
    rk = rk_ref[...]

    scores = _softmax(_leaky(jnp.dot(u, rc, preferred_element_type=jnp.float32)))
    ctx_agg = jnp.sum(scores[:, :, None] * ctx, axis=1)
    uf = _leaky(ctx_agg + u)
    imp = _softmax(_leaky(jnp.dot(uf, rk, preferred_element_type=jnp.float32)))
    es = jnp.sum(ent * uf[:, None, :], axis=2)
    out_ref[...] = jnp.sum(imp * es, axis=1)


_dense = pl.pallas_call(
    _dense_body,
    grid=(_B // _BLK,),
    in_specs=[
        pl.BlockSpec((_BLK, _D), lambda i: (i, 0)),
        pl.BlockSpec((_BLK, _NRC, _D), lambda i: (i, 0, 0)),
        pl.BlockSpec((_BLK, _NKC, _D), lambda i: (i, 0, 0)),
        pl.BlockSpec((_D, _NRC), lambda i: (0, 0)),
        pl.BlockSpec((_D, _NKC), lambda i: (0, 0)),
    ],
    out_specs=pl.BlockSpec((_BLK,), lambda i: (i,)),
    out_shape=jax.ShapeDtypeStruct((_B,), jnp.float32),
)


def kernel(user_factors, context_factors, entity_factors, relation_c, relation_k,
           user, item, contexts_index, entities_index):
    del item
    u_rows, c_rows, e_rows = _sc_gather()(
        user_factors, context_factors, entity_factors,
        user.astype(jnp.int32),
        contexts_index.reshape(-1).astype(jnp.int32),
        entities_index.reshape(-1).astype(jnp.int32),
    )
    return _dense(u_rows,
                  c_rows.reshape(_B, _NRC, _D),
                  e_rows.reshape(_B, _NKC, _D),
                  relation_c, relation_k)


# speedup vs baseline: 1.7356x; 1.2611x over previous
import functools

import jax
import jax.numpy as jnp
from jax import lax
from jax.experimental import pallas as pl
from jax.experimental.pallas import tpu as pltpu
from jax.experimental.pallas import tpu_sc as plsc

_B = 16384
_D = 64
_NK = 26

_NC, _NS = 2, 16
_NW = _NC * _NS
_BPW = _B // _NW

_CB = 8
_NCH = _BPW // _CB
_NPAIR = _NCH // 2
_CROWS = _CB * _NK

_F32 = jnp.float32


def _mesh():
    return plsc.VectorSubcoreMesh(core_axis_name="c", subcore_axis_name="s",
                                  num_cores=_NC, num_subcores=_NS)


def _wid():
    return lax.axis_index("s") * _NC + lax.axis_index("c")


def _sc_ugather_body(ufac_hbm, uidx2_hbm, upar_hbm, u_out,
                     idx_v, par_v, rows_v, out_v, sem):
    ub = _wid() * _BPW
    pltpu.sync_copy(uidx2_hbm.at[pl.ds(ub, _BPW)], idx_v)
    pltpu.sync_copy(upar_hbm.at[pl.ds(ub, _BPW)], par_v)
    pltpu.async_copy(ufac_hbm.at[idx_v], rows_v, sem).wait()

    def gbody(g, carry):
        pv = par_v[pl.ds(g * 16, 16)].astype(_F32)
        for i in range(16):
            b = g * 16 + i
            pf = pv[i]
            for dd in range(4):
                lo = rows_v[b, pl.ds(dd * 16, 16)]
                hi = rows_v[b, pl.ds(64 + dd * 16, 16)]
                out_v[pl.ds(b * _D + dd * 16, 16)] = lo + pf * (hi - lo)
        return carry

    lax.fori_loop(0, _BPW // 16, gbody, 0)
    pltpu.sync_copy(out_v, u_out.at[pl.ds(ub * _D, _BPW * _D)])


@functools.cache
def _sc_ugather():
    return pl.kernel(
        _sc_ugather_body,
        mesh=_mesh(),
        out_type=jax.ShapeDtypeStruct((_B * _D,), _F32),
        scratch_types=[
            pltpu.VMEM((_BPW,), jnp.int32),
            pltpu.VMEM((_BPW,), jnp.int32),
            pltpu.VMEM((_BPW, 2 * _D), _F32),
            pltpu.VMEM((_BPW * _D,), _F32),
            pltpu.SemaphoreType.DMA,
        ],
    )


def _sc_combine_body(table_hbm, idx_hbm, wlo_hbm, whi_hbm, out_hbm,
                     idx_all, wlo_all, whi_all, rows0, rows1, out_v,
                     sem0, sem1):
    bb = _wid() * _BPW

    pltpu.sync_copy(idx_hbm.at[pl.ds(bb * _NK, _BPW * _NK)], idx_all)
    pltpu.sync_copy(wlo_hbm.at[pl.ds(bb * 32, _BPW * 32)], wlo_all)
    pltpu.sync_copy(whi_hbm.at[pl.ds(bb * 32, _BPW * 32)], whi_all)

    def start(c, rows_v, sem):
        pltpu.async_copy(
            table_hbm.at[idx_all.at[pl.ds(c * _CROWS, _CROWS)]], rows_v, sem)

    def wait(c, rows_v, sem):
        pltpu.make_async_copy(
            table_hbm.at[idx_all.at[pl.ds(c * _CROWS, _CROWS)]], rows_v,
            sem).wait()

    def compute(c, rows_v):
        def bbody(b, carry):
            brow = c * _CB + b
            wl0 = wlo_all[pl.ds(brow * 32, 16)]
            wl1 = wlo_all[pl.ds(brow * 32 + 16, 16)]
            wh0 = whi_all[pl.ds(brow * 32, 16)]
            wh1 = whi_all[pl.ds(brow * 32 + 16, 16)]
            accs = [jnp.zeros((16,), _F32) for _ in range(4)]
            base = b * _NK
            for k in range(_NK):
                wl = wl0[k] if k < 16 else wl1[k - 16]
                wh = wh0[k] if k < 16 else wh1[k - 16]
                for dd in range(4):
                    lo = rows_v[base + k, pl.ds(dd * 16, 16)]
                    hi = rows_v[base + k, pl.ds(64 + dd * 16, 16)]
                    accs[dd] = accs[dd] + wl * lo + wh * hi
            for dd in range(4):
                out_v[pl.ds(b * _D + dd * 16, 16)] = accs[dd]
            return carry

        lax.fori_loop(0, _CB, bbody, 0)
        pltpu.sync_copy(out_v,
                        out_hbm.at[pl.ds((bb + c * _CB) * _D, _CB * _D)])

    start(0, rows0, sem0)

    def pair(j, carry):
        c0 = 2 * j
        start(c0 + 1, rows1, sem1)
        wait(c0, rows0, sem0)
        compute(c0, rows0)

        @pl.when(j < _NPAIR - 1)
        def _():
            start(c0 + 2, rows0, sem0)

        wait(c0 + 1, rows1, sem1)
        compute(c0 + 1, rows1)
        return carry

    lax.fori_loop(0, _NPAIR, pair, 0)


@functools.cache
def _sc_combine():
    return pl.kernel(
        _sc_combine_body,
        mesh=_mesh(),
        out_type=jax.ShapeDtypeStruct((_B * _D,), _F32),
        scratch_types=[
            pltpu.VMEM((_BPW * _NK,), jnp.int32),
            pltpu.VMEM((_BPW * 32,), _F32),
            pltpu.VMEM((_BPW * 32,), _F32),
            pltpu.VMEM((_CROWS, 2 * _D), _F32),
            pltpu.VMEM((_CROWS, 2 * _D), _F32),
            pltpu.VMEM((_CB * _D,), _F32),
            pltpu.SemaphoreType.DMA,
            pltpu.SemaphoreType.DMA,
        ],
    )


def _leaky(x):
    return jnp.where(x >= 0, x, 0.1 * x)


def _softmax(x):
    m = jnp.max(x, axis=1, keepdims=True)
    e = jnp.exp(x - m)
    return e / jnp.sum(e, axis=1, keepdims=True)


def _pad32(x):
    return jnp.pad(x, ((0, 0), (0, 32 - _NK)))


_TBLK = 2048


def _tc_scores_body(u_ref, rc_ref, cpar_ref, wlo_ref, whi_ref):
    s = _softmax(_leaky(
        jnp.dot(u_ref[...], rc_ref[...], preferred_element_type=_F32)))
    cp = cpar_ref[...]
    wlo_ref[...] = _pad32(s * (1.0 - cp))
    whi_ref[...] = _pad32(s * cp)


_tc_scores = pl.pallas_call(
    _tc_scores_body,
    grid=(_B // _TBLK,),
    in_specs=[
        pl.BlockSpec((_TBLK, _D), lambda i: (i, 0)),
        pl.BlockSpec((_D, _NK), lambda i: (0, 0)),
        pl.BlockSpec((_TBLK, _NK), lambda i: (i, 0)),
    ],
    out_specs=[
        pl.BlockSpec((_TBLK, 32), lambda i: (i, 0)),
        pl.BlockSpec((_TBLK, 32), lambda i: (i, 0)),
    ],
    out_shape=[
        jax.ShapeDtypeStruct((_B, 32), _F32),
        jax.ShapeDtypeStruct((_B, 32), _F32),
    ],
)


def _tc_mid_body(agg_ref, u_ref, rk_ref, epar_ref, uf_ref, ilo_ref, ihi_ref):
    uf = _leaky(agg_ref[...] + u_ref[...])
    uf_ref[...] = uf
    imp = _softmax(_leaky(
        jnp.dot(uf, rk_ref[...], preferred_element_type=_F32)))
    ep = epar_ref[...]
    ilo_ref[...] = _pad32(imp * (1.0 - ep))
    ihi_ref[...] = _pad32(imp * ep)


_tc_mid = pl.pallas_call(
    _tc_mid_body,
    grid=(_B // _TBLK,),
    in_specs=[
        pl.BlockSpec((_TBLK, _D), lambda i: (i, 0)),
        pl.BlockSpec((_TBLK, _D), lambda i: (i, 0)),
        pl.BlockSpec((_D, _NK), lambda i: (0, 0)),
        pl.BlockSpec((_TBLK, _NK), lambda i: (i, 0)),
    ],
    out_specs=[
        pl.BlockSpec((_TBLK, _D), lambda i: (i, 0)),
        pl.BlockSpec((_TBLK, 32), lambda i: (i, 0)),
        pl.BlockSpec((_TBLK, 32), lambda i: (i, 0)),
    ],
    out_shape=[
        jax.ShapeDtypeStruct((_B, _D), _F32),
        jax.ShapeDtypeStruct((_B, 32), _F32),
        jax.ShapeDtypeStruct((_B, 32), _F32),
    ],
)


def _tc_final_body(agg_ref, uf_ref, out_ref):
    out_ref[...] = jnp.sum(agg_ref[...] * uf_ref[...], axis=1)


_tc_final = pl.pallas_call(
    _tc_final_body,
    grid=(_B // _TBLK,),
    in_specs=[
        pl.BlockSpec((_TBLK, _D), lambda i: (i, 0)),
        pl.BlockSpec((_TBLK, _D), lambda i: (i, 0)),
    ],
    out_specs=pl.BlockSpec((_TBLK,), lambda i: (i,)),
    out_shape=jax.ShapeDtypeStruct((_B,), _F32),
)


def kernel(user_factors, context_factors, entity_factors, relation_c, relation_k,
           user, item, contexts_index, entities_index):
    del item
    ufac2 = user_factors.reshape(-1, 2 * _D)
    cfac2 = context_factors.reshape(-1, 2 * _D)
    efac2 = entity_factors.reshape(-1, 2 * _D)

    uidx = user.astype(jnp.int32)
    cidx = contexts_index.reshape(-1).astype(jnp.int32)
    eidx = entities_index.reshape(-1).astype(jnp.int32)

    cparf = (contexts_index & 1).astype(_F32)
    eparf = (entities_index & 1).astype(_F32)

    u = _sc_ugather()(ufac2, uidx >> 1, uidx & 1).reshape(_B, _D)
    wlo, whi = _tc_scores(u, relation_c, cparf)
    ctx_agg = _sc_combine()(cfac2, cidx >> 1,
                            wlo.reshape(-1), whi.reshape(-1)).reshape(_B, _D)
    uf, ilo, ihi = _tc_mid(ctx_agg, u, relation_k, eparf)
    ent_agg = _sc_combine()(efac2, eidx >> 1,
                            ilo.reshape(-1), ihi.reshape(-1)).reshape(_B, _D)
    return _tc_final(ent_agg, uf)
